# baseline (speedup 1.0000x reference)
import jax
import jax.numpy as jnp
from jax.experimental import pallas as pl
from jax.experimental.pallas import tpu as pltpu

_N = 4096
_BR = 1024
_CH = 1024
_NB = _N // _BR
_LOG2E = 1.4426950408889634


def _prologue(h, w, af_col, as_row, whb_s, ea_s, eb_s, ca_s, cb_s):
    wh = jnp.dot(h, w, preferred_element_type=jnp.float32)
    f = w.shape[1]
    whb0 = wh.astype(jnp.bfloat16)
    whb = whb0
    if f < 128:
        whb = jnp.concatenate(
            [whb0, jnp.zeros((_N, 128 - f), jnp.bfloat16)], axis=1)
    whb_s[:, 0:128] = whb
    whb_s[:, 128:256] = jnp.ones((_N, 128), jnp.bfloat16)
    e1 = jnp.dot(wh, af_col, preferred_element_type=jnp.float32)
    e2r = jax.lax.dot_general(
        as_row, wh, (((1,), (1,)), ((), ())),
        preferred_element_type=jnp.float32)
    e2m = jnp.max(e2r)
    tm = e1 + e2m
    c = jnp.maximum(tm, 0.2 * tm)
    ea_s[...] = ((e1 - c) * _LOG2E).astype(jnp.bfloat16)
    eb_s[...] = ((0.2 * e1 - c) * _LOG2E).astype(jnp.bfloat16)
    ca_s[...] = (e2r * _LOG2E).astype(jnp.bfloat16)
    cb_s[...] = (e2r * (0.2 * _LOG2E)).astype(jnp.bfloat16)


def _attention(adj_chunk, r, whb_s, ea_s, eb_s, ca_s, cb_s):
    rows = pl.ds(r * _BR, _BR)
    ea_b = ea_s[rows, :]
    eb_b = eb_s[rows, :]
    acc = None
    for k in range(_N // _CH):
        cols = pl.ds(k * _CH, _CH)
        adjk = adj_chunk(k)
        va = ea_b + ca_s[:, cols]
        vb = eb_b + cb_s[:, cols]
        pk = jnp.exp2(jnp.maximum(va, vb)) * adjk
        ok = jnp.dot(pk, whb_s[cols, :], preferred_element_type=jnp.float32)
        acc = ok if acc is None else acc + ok
    return acc[:, 0:128] / acc[:, 128:256]


def _body(x_ref, w1_ref, a1f_ref, a1s_ref, w2_ref, a2f_ref, a2s_ref,
          wl_ref, bl_ref, adj_ref, out_ref,
          whb_s, ea_s, eb_s, ca_s, cb_s, h1_s):
    i = pl.program_id(0)

    @pl.when(i == 0)
    def _init1():
        _prologue(x_ref[...], w1_ref[...], a1f_ref[...], a1s_ref[...],
                  whb_s, ea_s, eb_s, ca_s, cb_s)

    @pl.when(i < _NB)
    def _phase1():
        rows = pl.ds(i * _BR, _BR)

        def chunk(k):
            return adj_ref[:, pl.ds(k * _CH, _CH)].astype(jnp.bfloat16)

        o = _attention(chunk, i, whb_s, ea_s, eb_s, ca_s, cb_s)
        h1_s[rows, :] = jnp.where(o > 0.0, o, jnp.exp(o) - 1.0)

    @pl.when(i == _NB)
    def _init2():
        _prologue(h1_s[...], w2_ref[...], a2f_ref[...], a2s_ref[...],
                  whb_s, ea_s, eb_s, ca_s, cb_s)

    @pl.when(i >= _NB)
    def _phase2():
        r = i - _NB

        def chunk(k):
            return adj_ref[:, pl.ds(k * _CH, _CH)].astype(jnp.bfloat16)

        o = _attention(chunk, r, whb_s, ea_s, eb_s, ca_s, cb_s)
        h2 = jax.nn.sigmoid(o)
        out_ref[...] = (
            jnp.dot(h2, wl_ref[...], preferred_element_type=jnp.float32)
            + bl_ref[...])


def kernel(x, adj, W1, a1, W2, a2, Wl, bl):
    t = x.shape[1]
    x_pad = jnp.pad(x, ((0, 0), (0, 128 - t)))
    w1_pad = jnp.pad(W1, ((0, 128 - t), (0, 0)))
    wl_pad = jnp.pad(Wl, ((0, 128 - Wl.shape[0]), (0, 128 - Wl.shape[1])))
    bl_pad = jnp.pad(bl, (0, 128 - bl.shape[0])).reshape(1, 128)
    cst = lambda i: (0, 0)
    out = pl.pallas_call(
        _body,
        grid=(2 * _NB,),
        in_specs=[
            pl.BlockSpec((_N, 128), cst),
            pl.BlockSpec((128, 128), cst),
            pl.BlockSpec((128, 1), cst),
            pl.BlockSpec((1, 128), cst),
            pl.BlockSpec((128, 64), cst),
            pl.BlockSpec((64, 1), cst),
            pl.BlockSpec((1, 64), cst),
            pl.BlockSpec((128, 128), cst),
            pl.BlockSpec((1, 128), cst),
            pl.BlockSpec((_BR, _N), lambda i: (jax.lax.rem(i, _NB), 0)),
        ],
        out_specs=pl.BlockSpec(
            (_BR, 128), lambda i: (jnp.maximum(i - _NB, 0), 0)),
        out_shape=jax.ShapeDtypeStruct((_N, 128), jnp.float32),
        scratch_shapes=[
            pltpu.VMEM((_N, 256), jnp.bfloat16),
            pltpu.VMEM((_N, 1), jnp.bfloat16),
            pltpu.VMEM((_N, 1), jnp.bfloat16),
            pltpu.VMEM((1, _N), jnp.bfloat16),
            pltpu.VMEM((1, _N), jnp.bfloat16),
            pltpu.VMEM((_N, 128), jnp.float32),
        ],
    )(x_pad, w1_pad,
      a1[:128].reshape(128, 1), a1[128:].reshape(1, 128),
      W2, a2[:64].reshape(64, 1), a2[64:].reshape(1, 64),
      wl_pad, bl_pad, adj)
    return out[:, :Wl.shape[1]]


# speedup vs baseline: 1.0235x; 1.0235x over previous
import jax
import jax.numpy as jnp
from jax.experimental import pallas as pl
from jax.experimental.pallas import tpu as pltpu

_N = 4096
_BR = 512
_CH = 1024
_NB = _N // _BR
_LOG2E = 1.4426950408889634


def _prologue(h, w, af_col, as_row, whb_s, ea_s, eb_s, ca_s, cb_s):
    wh = jnp.dot(h, w, preferred_element_type=jnp.float32)
    f = w.shape[1]
    whb0 = wh.astype(jnp.bfloat16)
    whb = whb0
    if f < 128:
        whb = jnp.concatenate(
            [whb0, jnp.zeros((_N, 128 - f), jnp.bfloat16)], axis=1)
    whb_s[:, 0:128] = whb
    whb_s[:, 128:256] = jnp.ones((_N, 128), jnp.bfloat16)
    e1r = jax.lax.dot_general(
        af_col, wh, (((0,), (1,)), ((), ())),
        preferred_element_type=jnp.float32)
    e2r = jax.lax.dot_general(
        as_row, wh, (((1,), (1,)), ((), ())),
        preferred_element_type=jnp.float32)
    e2m = jnp.max(e2r)
    tm = e1r + e2m
    c = jnp.maximum(tm, 0.2 * tm)
    ear = ((e1r - c) * _LOG2E).astype(jnp.bfloat16)
    ebr = ((0.2 * e1r - c) * _LOG2E).astype(jnp.bfloat16)
    ea_s[...] = jnp.transpose(ear)
    eb_s[...] = jnp.transpose(ebr)
    ca_s[...] = (e2r * _LOG2E).astype(jnp.bfloat16)
    cb_s[...] = (e2r * (0.2 * _LOG2E)).astype(jnp.bfloat16)


def _attention(adj_chunk, r, whb_s, ea_s, eb_s, ca_s, cb_s):
    rows = pl.ds(r * _BR, _BR)
    ea_b = ea_s[rows, :]
    eb_b = eb_s[rows, :]
    acc = None
    for k in range(_N // _CH):
        cols = pl.ds(k * _CH, _CH)
        adjk = adj_chunk(k)
        va = ea_b + ca_s[:, cols]
        vb = eb_b + cb_s[:, cols]
        pk = jnp.exp2(jnp.maximum(va, vb)) * adjk
        ok = jnp.dot(pk, whb_s[cols, :], preferred_element_type=jnp.float32)
        acc = ok if acc is None else acc + ok
    return acc[:, 0:128] / acc[:, 128:256]


def _body(x_ref, w1_ref, a1f_ref, a1s_ref, w2_ref, a2f_ref, a2s_ref,
          wl_ref, bl_ref, adj_ref, out_ref,
          whb_s, ea_s, eb_s, ca_s, cb_s, h1_s):
    i = pl.program_id(0)

    @pl.when(i == 0)
    def _init1():
        _prologue(x_ref[...], w1_ref[...], a1f_ref[...], a1s_ref[...],
                  whb_s, ea_s, eb_s, ca_s, cb_s)

    @pl.when(i < _NB)
    def _phase1():
        rows = pl.ds(i * _BR, _BR)

        def chunk(k):
            return adj_ref[:, pl.ds(k * _CH, _CH)].astype(jnp.bfloat16)

        o = _attention(chunk, i, whb_s, ea_s, eb_s, ca_s, cb_s)
        h1_s[rows, :] = jnp.where(o > 0.0, o, jnp.exp(o) - 1.0)

    @pl.when(i == _NB)
    def _init2():
        _prologue(h1_s[...], w2_ref[...], a2f_ref[...], a2s_ref[...],
                  whb_s, ea_s, eb_s, ca_s, cb_s)

    @pl.when(i >= _NB)
    def _phase2():
        r = i - _NB

        def chunk(k):
            return adj_ref[:, pl.ds(k * _CH, _CH)].astype(jnp.bfloat16)

        o = _attention(chunk, r, whb_s, ea_s, eb_s, ca_s, cb_s)
        h2 = jax.nn.sigmoid(o)
        out_ref[...] = (
            jnp.dot(h2, wl_ref[...], preferred_element_type=jnp.float32)
            + bl_ref[...])


def kernel(x, adj, W1, a1, W2, a2, Wl, bl):
    t = x.shape[1]
    x_pad = jnp.pad(x, ((0, 0), (0, 128 - t)))
    w1_pad = jnp.pad(W1, ((0, 128 - t), (0, 0)))
    wl_pad = jnp.pad(Wl, ((0, 128 - Wl.shape[0]), (0, 128 - Wl.shape[1])))
    bl_pad = jnp.pad(bl, (0, 128 - bl.shape[0])).reshape(1, 128)
    cst = lambda i: (0, 0)
    out = pl.pallas_call(
        _body,
        grid=(2 * _NB,),
        in_specs=[
            pl.BlockSpec((_N, 128), cst),
            pl.BlockSpec((128, 128), cst),
            pl.BlockSpec((128, 1), cst),
            pl.BlockSpec((1, 128), cst),
            pl.BlockSpec((128, 64), cst),
            pl.BlockSpec((64, 1), cst),
            pl.BlockSpec((1, 64), cst),
            pl.BlockSpec((128, 128), cst),
            pl.BlockSpec((1, 128), cst),
            pl.BlockSpec((_BR, _N), lambda i: (jax.lax.rem(i, _NB), 0)),
        ],
        out_specs=pl.BlockSpec(
            (_BR, 128), lambda i: (jnp.maximum(i - _NB, 0), 0)),
        out_shape=jax.ShapeDtypeStruct((_N, 128), jnp.float32),
        scratch_shapes=[
            pltpu.VMEM((_N, 256), jnp.bfloat16),
            pltpu.VMEM((_N, 1), jnp.bfloat16),
            pltpu.VMEM((_N, 1), jnp.bfloat16),
            pltpu.VMEM((1, _N), jnp.bfloat16),
            pltpu.VMEM((1, _N), jnp.bfloat16),
            pltpu.VMEM((_N, 128), jnp.float32),
        ],
    )(x_pad, w1_pad,
      a1[:128].reshape(128, 1), a1[128:].reshape(1, 128),
      W2, a2[:64].reshape(64, 1), a2[64:].reshape(1, 64),
      wl_pad, bl_pad, adj)
    return out[:, :Wl.shape[1]]
